# baseline (speedup 1.0000x reference)
import functools

import jax
import jax.numpy as jnp
from jax import lax
from jax.experimental import pallas as pl
from jax.experimental.pallas import tpu as pltpu
from jax.experimental.pallas import tpu_sc as plsc

CH = 128
NBUF = 5


def kernel(token_ids, weight):
    orig_shape = token_ids.shape
    D = weight.shape[1]
    B = token_ids.size

    info = plsc.get_sparse_core_info()
    NC, NS = info.num_cores, info.num_subcores
    NW = NC * NS
    rows_per_w = B // (NW * CH)
    n_outer = rows_per_w // NBUF
    idx3d = token_ids.reshape(NW, rows_per_w, CH).astype(jnp.int32)

    mesh = plsc.VectorSubcoreMesh(core_axis_name="c", subcore_axis_name="s")

    @functools.partial(
        pl.kernel,
        mesh=mesh,
        compiler_params=pltpu.CompilerParams(use_tc_tiling_on_sc=False),
        out_type=jax.ShapeDtypeStruct((B, D), jnp.float32),
        scratch_types=[
            pltpu.VMEM((rows_per_w, CH), jnp.int32),
            pltpu.VMEM((CH, D), jnp.float32),
        ],
    )
    def k(idx_hbm, table_hbm, out_hbm, idx_v, rows_v):
        wid = lax.axis_index("s") * NC + lax.axis_index("c")
        base = wid * rows_per_w * CH
        pltpu.sync_copy(idx_hbm.at[wid], idx_v)

        def body(g, _):
            pltpu.sync_copy(table_hbm.at[idx_v.at[g]], rows_v)
            pltpu.sync_copy(rows_v, out_hbm.at[pl.ds(base + g * CH, CH)])
            return 0

        lax.fori_loop(0, rows_per_w, body, 0)

    out = k(idx3d, weight)
    return out.reshape(*orig_shape, D)


# speedup vs baseline: 1.0449x; 1.0449x over previous
import functools

import jax
import jax.numpy as jnp
from jax import lax
from jax.experimental import pallas as pl
from jax.experimental.pallas import tpu as pltpu
from jax.experimental.pallas import tpu_sc as plsc

CH = 128
NBUF = 5


def kernel(token_ids, weight):
    orig_shape = token_ids.shape
    D = weight.shape[1]
    B = token_ids.size

    info = plsc.get_sparse_core_info()
    NC, NS = info.num_cores, info.num_subcores
    NW = NC * NS
    rows_per_w = B // (NW * CH)
    n_outer = rows_per_w // NBUF
    idx3d = token_ids.reshape(NW, rows_per_w, CH).astype(jnp.int32)

    mesh = plsc.VectorSubcoreMesh(core_axis_name="c", subcore_axis_name="s")

    @functools.partial(
        pl.kernel,
        mesh=mesh,
        compiler_params=pltpu.CompilerParams(use_tc_tiling_on_sc=False),
        out_type=jax.ShapeDtypeStruct((B, D), jnp.float32),
        scratch_types=[
            pltpu.VMEM((rows_per_w, CH), jnp.int32),
            pltpu.VMEM((NBUF, CH, D), jnp.float32),
            pltpu.SemaphoreType.DMA((NBUF,)),
        ],
    )
    def k(idx_hbm, table_hbm, out_hbm, idx_v, rows_v, sems):
        wid = lax.axis_index("s") * NC + lax.axis_index("c")
        base = wid * rows_per_w * CH
        pltpu.sync_copy(idx_hbm.at[wid], idx_v)
        for b in range(NBUF):
            pltpu.async_copy(table_hbm.at[idx_v.at[b]], rows_v.at[b], sems.at[b])

        def wait_gather(b):
            pltpu.make_async_copy(
                table_hbm.at[idx_v.at[0]], rows_v.at[b], sems.at[b]
            ).wait()

        def body(it, _):
            j = it * NBUF
            for b in range(NBUF):
                g = j + b
                wait_gather(b)
                pltpu.sync_copy(
                    rows_v.at[b], out_hbm.at[pl.ds(base + g * CH, CH)]
                )
                pltpu.async_copy(
                    table_hbm.at[idx_v.at[g + NBUF]], rows_v.at[b], sems.at[b]
                )
            return 0

        lax.fori_loop(0, n_outer - 1, body, 0)
        for b in range(NBUF):
            g = (n_outer - 1) * NBUF + b
            wait_gather(b)
            pltpu.sync_copy(rows_v.at[b], out_hbm.at[pl.ds(base + g * CH, CH)])

    out = k(idx3d, weight)
    return out.reshape(*orig_shape, D)
